# baseline (speedup 1.0000x reference)
import functools

import jax
import jax.numpy as jnp
from jax import lax
from jax.experimental import pallas as pl
from jax.experimental.pallas import tpu as pltpu
from jax.experimental.pallas import tpu_sc as plsc

N = 10000
NP = 10240
E = 160000
D_IN = 256
INNER = 1024
D_H = 512
CW = 128
NC, NS = 2, 16
NW = NC * NS
VREGS = 313
EPT = VREGS * 16
EP = NW * EPT
NWIN = 4
WR = NP // NWIN
SPT = WR // NS
B2 = 32
NB3 = 158
B = 125
NB = (E // NW) // B
ROWS_PER_TILE = NP // NS

NBK = 1024
GRID = NP // NBK

_SC_MESH = plsc.VectorSubcoreMesh(core_axis_name="c", subcore_axis_name="s")


@functools.partial(
    pl.kernel,
    out_type=jax.ShapeDtypeStruct((NC, NP, CW), jnp.float32),
    mesh=_SC_MESH,
    scratch_types=[
        pltpu.VMEM((NB, B), jnp.int32),
        pltpu.VMEM((B, CW), jnp.float32),
        pltpu.VMEM_SHARED((NP, CW), jnp.float32),
    ],
)
def _sc_deg(dst_hbm, ones_hbm, zeros_hbm, out_hbm, dstv, onesv, shared):
    c = lax.axis_index("c")
    s = lax.axis_index("s")
    w = c * NS + s
    pltpu.sync_copy(dst_hbm.at[w], dstv)
    pltpu.sync_copy(ones_hbm, onesv)
    sl = pl.ds(s * ROWS_PER_TILE, ROWS_PER_TILE)
    pltpu.sync_copy(zeros_hbm, shared.at[sl])
    plsc.subcore_barrier()

    def body(b, carry):
        pltpu.sync_copy(onesv, shared.at[dstv.at[b]], add=True)
        return carry

    lax.fori_loop(0, NB, body, 0)
    plsc.subcore_barrier()
    pltpu.sync_copy(shared.at[sl], out_hbm.at[c].at[sl])


@functools.partial(
    pl.kernel,
    out_type=[
        jax.ShapeDtypeStruct((NW, NWIN, NB3, B2), jnp.int32),
        jax.ShapeDtypeStruct((NW, NWIN, NB3, B2), jnp.int32),
        jax.ShapeDtypeStruct((NW, 16), jnp.int32),
    ],
    mesh=_SC_MESH,
    scratch_types=[
        pltpu.VMEM((VREGS, 16), jnp.int32),
        pltpu.VMEM((VREGS, 16), jnp.int32),
        pltpu.VMEM((NWIN, NB3, B2), jnp.int32),
        pltpu.VMEM((NWIN, NB3, B2), jnp.int32),
        pltpu.VMEM((16,), jnp.int32),
    ],
    compiler_params=pltpu.CompilerParams(
        needs_layout_passes=False, use_tc_tiling_on_sc=False),
)
def _sc_part(src_hbm, dst_hbm, csrc_hbm, cdst_hbm, cnt_hbm,
             srcst, dstst, lsrc, ldst, cntv):
    c = lax.axis_index("c")
    s = lax.axis_index("s")
    w = c * NS + s
    pltpu.sync_copy(src_hbm.at[w], srcst)
    pltpu.sync_copy(dst_hbm.at[w], dstst)
    iota = lax.iota(jnp.int32, 16)

    def vbody(k, offs):
        sv = srcst[k]
        dv = dstst[k]
        new = []
        for win in range(NWIN):
            m = (dv >= win * WR) & (dv < (win + 1) * WR)
            cum = plsc.cumsum(m.astype(jnp.int32))
            pos = offs[win] + cum - 1
            row = lax.shift_right_logical(pos, 5)
            col = lax.bitwise_and(pos, 31)
            plsc.store_scatter(lsrc.at[win], [row, col], sv, mask=m)
            plsc.store_scatter(ldst.at[win], [row, col], dv - win * WR, mask=m)
            new.append(offs[win] + jnp.max(cum))
        return tuple(new)

    z = jnp.int32(0)
    offs = lax.fori_loop(0, VREGS, vbody, (z, z, z, z))
    for win in range(NWIN):
        for t in range(2):
            pos = jnp.full((16,), offs[win] + t * 16, jnp.int32) + iota
            row = lax.shift_right_logical(pos, 5)
            col = lax.bitwise_and(pos, 31)
            plsc.store_scatter(lsrc.at[win], [row, col], N + iota)
            plsc.store_scatter(ldst.at[win], [row, col], iota)
    cv = jnp.where(iota == 0, offs[0], 0)
    for win in range(1, NWIN):
        cv = jnp.where(iota == win, offs[win], cv)
    cntv[...] = cv
    pltpu.sync_copy(lsrc, csrc_hbm.at[w])
    pltpu.sync_copy(ldst, cdst_hbm.at[w])
    pltpu.sync_copy(cntv, cnt_hbm.at[w])


@functools.partial(
    pl.kernel,
    out_type=jax.ShapeDtypeStruct((NP, D_H), jnp.float32),
    mesh=_SC_MESH,
    scratch_types=[
        pltpu.VMEM((NB3, B2), jnp.int32),
        pltpu.VMEM((NB3, B2), jnp.int32),
        pltpu.VMEM((16,), jnp.int32),
        pltpu.VMEM((B2, D_H), jnp.float32),
        pltpu.VMEM((B2, D_H), jnp.float32),
        pltpu.VMEM_SHARED((WR, D_H), jnp.float32),
        pltpu.SemaphoreType.DMA,
        pltpu.SemaphoreType.DMA,
    ],
    compiler_params=pltpu.CompilerParams(
        needs_layout_passes=False, use_tc_tiling_on_sc=False),
)
def _sc_scatter(csrc_hbm, cdst_hbm, cnt_hbm, f_hbm, zeros_hbm, out_hbm,
                csv, cdv, cntv, rows0, rows1, shared, sem0, sem1):
    c = lax.axis_index("c")
    s = lax.axis_index("s")
    iota = lax.iota(jnp.int32, 16)
    lstripe = pl.ds(s * SPT, SPT)
    for p in range(NWIN // NC):
        wi = p * NC + c
        pltpu.sync_copy(zeros_hbm, shared.at[lstripe])
        plsc.subcore_barrier()
        for half in range(2):
            wsrc = 2 * s + half
            pltpu.sync_copy(cnt_hbm.at[wsrc], cntv)
            pltpu.sync_copy(csrc_hbm.at[wsrc].at[wi], csv)
            pltpu.sync_copy(cdst_hbm.at[wsrc].at[wi], cdv)
            cnt = jnp.max(jnp.where(iota == wi, cntv[...], 0))
            nb = (cnt + B2 - 1) // B2

            @pl.when(nb > 0)
            def _():
                pltpu.async_copy(f_hbm.at[csv.at[0]], rows0, sem0)

            def body(b, carry):
                def step(rj, rk, sj, sk):
                    pltpu.make_async_copy(f_hbm.at[csv.at[b]], rj, sj).wait()

                    @pl.when(b + 1 < nb)
                    def _():
                        pltpu.async_copy(f_hbm.at[csv.at[b + 1]], rk, sk)

                    pltpu.sync_copy(rj, shared.at[cdv.at[b]], add=True)

                @pl.when(lax.rem(b, 2) == 0)
                def _():
                    step(rows0, rows1, sem0, sem1)

                @pl.when(lax.rem(b, 2) == 1)
                def _():
                    step(rows1, rows0, sem1, sem0)

                return carry

            lax.fori_loop(0, nb, body, 0)
        plsc.subcore_barrier()
        pltpu.sync_copy(shared.at[lstripe],
                        out_hbm.at[pl.ds(wi * WR + s * SPT, SPT)])


def _ln(v, g, b):
    m = jnp.mean(v, axis=-1, keepdims=True)
    var = jnp.mean((v - m) ** 2, axis=-1, keepdims=True)
    return (v - m) * lax.rsqrt(var + 1e-5) * g + b


def _ff_body(x_ref, w1_ref, b1_ref, g1_ref, be1_ref, w2_ref, b2_ref,
             g2_ref, be2_ref, o_ref):
    h = jnp.dot(x_ref[...], w1_ref[...], preferred_element_type=jnp.float32)
    h = h + b1_ref[...]
    h = h * jax.nn.sigmoid(h)
    h = _ln(h, g1_ref[...], be1_ref[...])
    h = jnp.dot(h, w2_ref[...], preferred_element_type=jnp.float32)
    h = h + b2_ref[...]
    o_ref[...] = _ln(h, g2_ref[...], be2_ref[...])


def _tc_ff(x, p):
    full = lambda shape: pl.BlockSpec(shape, lambda i: (0,) * len(shape))
    return pl.pallas_call(
        _ff_body,
        grid=(GRID,),
        in_specs=[
            pl.BlockSpec((NBK, D_IN), lambda i: (i, 0)),
            full((D_IN, INNER)),
            full((1, INNER)), full((1, INNER)), full((1, INNER)),
            full((INNER, D_H)),
            full((1, D_H)), full((1, D_H)), full((1, D_H)),
        ],
        out_specs=pl.BlockSpec((NBK, D_H), lambda i: (i, 0)),
        out_shape=jax.ShapeDtypeStruct((NP, D_H), jnp.float32),
    )(x, p['W1'], p['b1'].reshape(1, -1), p['ln1_g'].reshape(1, -1),
      p['ln1_b'].reshape(1, -1), p['W2'], p['b2'].reshape(1, -1),
      p['ln2_g'].reshape(1, -1), p['ln2_b'].reshape(1, -1))


def _row_mask(col):
    rid = (pl.program_id(0) * NBK
           + lax.broadcasted_iota(jnp.int32, (NBK, 1), 0))
    return jnp.where(rid < N, col, 0.0)


def _prep_body(deg_ref, h_ref, norm_ref, f_ref):
    deg = deg_ref[0, :, 0:1] + deg_ref[1, :, 0:1]
    norm = lax.rsqrt(jnp.maximum(deg, 1.0))
    norm_ref[...] = jnp.broadcast_to(norm, (NBK, CW))
    f_ref[...] = h_ref[...] * _row_mask(norm)


def _tc_prep(deg_parts, h):
    return pl.pallas_call(
        _prep_body,
        grid=(GRID,),
        in_specs=[
            pl.BlockSpec((NC, NBK, CW), lambda i: (0, i, 0)),
            pl.BlockSpec((NBK, D_H), lambda i: (i, 0)),
        ],
        out_specs=[
            pl.BlockSpec((NBK, CW), lambda i: (i, 0)),
            pl.BlockSpec((NBK, D_H), lambda i: (i, 0)),
        ],
        out_shape=[
            jax.ShapeDtypeStruct((NP, CW), jnp.float32),
            jax.ShapeDtypeStruct((NP, D_H), jnp.float32),
        ],
    )(deg_parts, h)


def _gcn_hnew(agg_ref, h_ref, norm_ref, wg_ref, bg_ref, wr_ref, br_ref):
    norm = norm_ref[:, 0:1]
    f = agg_ref[...] * norm
    hn = (jnp.dot(f, wg_ref[...], preferred_element_type=jnp.float32)
          + jnp.dot(h_ref[...], wr_ref[...], preferred_element_type=jnp.float32)
          + bg_ref[...] + br_ref[...])
    return hn, norm


def _gcn_body(agg_ref, h_ref, norm_ref, wg_ref, bg_ref, wr_ref, br_ref,
              h_out, f_out):
    hn, norm = _gcn_hnew(agg_ref, h_ref, norm_ref, wg_ref, bg_ref,
                         wr_ref, br_ref)
    h_out[...] = hn
    f_out[...] = hn * _row_mask(norm)


def _gcn_final_body(agg_ref, h_ref, norm_ref, wg_ref, bg_ref, wr_ref, br_ref,
                    o_ref):
    hn, _ = _gcn_hnew(agg_ref, h_ref, norm_ref, wg_ref, bg_ref,
                      wr_ref, br_ref)
    rid = (pl.program_id(0) * NBK
           + lax.broadcasted_iota(jnp.int32, (NBK, 1), 0))
    hn = jnp.where(rid < N, hn, -jnp.inf)

    @pl.when(pl.program_id(0) == 0)
    def _():
        o_ref[...] = jnp.full((1, D_H), -jnp.inf, dtype=jnp.float32)

    o_ref[...] = jnp.maximum(o_ref[...], jnp.max(hn, axis=0, keepdims=True))


def _gcn_specs(final):
    in_specs = [
        pl.BlockSpec((NBK, D_H), lambda i: (i, 0)),
        pl.BlockSpec((NBK, D_H), lambda i: (i, 0)),
        pl.BlockSpec((NBK, CW), lambda i: (i, 0)),
        pl.BlockSpec((D_H, D_H), lambda i: (0, 0)),
        pl.BlockSpec((1, D_H), lambda i: (0, 0)),
        pl.BlockSpec((D_H, D_H), lambda i: (0, 0)),
        pl.BlockSpec((1, D_H), lambda i: (0, 0)),
    ]
    if final:
        return in_specs, pl.BlockSpec((1, D_H), lambda i: (0, 0)), \
            jax.ShapeDtypeStruct((1, D_H), jnp.float32)
    return in_specs, [
        pl.BlockSpec((NBK, D_H), lambda i: (i, 0)),
        pl.BlockSpec((NBK, D_H), lambda i: (i, 0)),
    ], [
        jax.ShapeDtypeStruct((NP, D_H), jnp.float32),
        jax.ShapeDtypeStruct((NP, D_H), jnp.float32),
    ]


def _tc_gcn(agg, h, norm128, wg, bg, wr, br, final):
    in_specs, out_specs, out_shape = _gcn_specs(final)
    body = _gcn_final_body if final else _gcn_body
    return pl.pallas_call(
        body,
        grid=(GRID,),
        in_specs=in_specs,
        out_specs=out_specs,
        out_shape=out_shape,
    )(agg, h, norm128, wg, bg.reshape(1, -1), wr, br.reshape(1, -1))


def kernel(x, edge_index, params):
    ei = edge_index.astype(jnp.int32)
    src = ei[0]
    dst = ei[1]
    dst3 = dst.reshape(NW, NB, B)
    padn = EP - E
    pad_idx = jnp.arange(padn, dtype=jnp.int32)
    src_p = jnp.concatenate([src, N + pad_idx % (NP - N)]).reshape(
        NW, VREGS, 16)
    dst_p = jnp.concatenate([dst, pad_idx % CW]).reshape(NW, VREGS, 16)
    x = jnp.pad(x, ((0, NP - N), (0, 0)))
    ones128 = jnp.ones((B, CW), jnp.float32)
    zeros_deg = jnp.zeros((ROWS_PER_TILE, CW), jnp.float32)
    zeros_win = jnp.zeros((SPT, D_H), jnp.float32)

    csrc, cdst, cnt = _sc_part(src_p, dst_p)
    zeros_deg = zeros_deg + (cnt[0, 0] * 0).astype(jnp.float32)
    deg_parts = _sc_deg(dst3, ones128, zeros_deg)
    h = _tc_ff(x, params)
    norm128, f = _tc_prep(deg_parts, h)
    out = None
    for i in range(4):
        agg = _sc_scatter(csrc, cdst, cnt, f, zeros_win)
        res = _tc_gcn(agg, h, norm128,
                      params['gcn%d_W' % i], params['gcn%d_b' % i],
                      params['res%d_W' % i], params['res%d_b' % i],
                      final=(i == 3))
        if i < 3:
            h, f = res
        else:
            out = res
    return out


# speedup vs baseline: 1.0029x; 1.0029x over previous
import functools

import jax
import jax.numpy as jnp
from jax import lax
from jax.experimental import pallas as pl
from jax.experimental.pallas import tpu as pltpu
from jax.experimental.pallas import tpu_sc as plsc

N = 10000
NP = 10240
E = 160000
D_IN = 256
INNER = 1024
D_H = 512
CW = 128
NC, NS = 2, 16
NW = NC * NS
VREGS = 313
EPT = VREGS * 16
EP = NW * EPT
NWIN = 8
WR = NP // NWIN
SPT = WR // NS
B2 = 64
B2SH = 6
NB3 = 80
B = 125
NB = (E // NW) // B
ROWS_PER_TILE = NP // NS

NBK = 1024
GRID = NP // NBK

_SC_MESH = plsc.VectorSubcoreMesh(core_axis_name="c", subcore_axis_name="s")


@functools.partial(
    pl.kernel,
    out_type=jax.ShapeDtypeStruct((NC, NP, CW), jnp.float32),
    mesh=_SC_MESH,
    scratch_types=[
        pltpu.VMEM((NB, B), jnp.int32),
        pltpu.VMEM((B, CW), jnp.float32),
        pltpu.VMEM_SHARED((NP, CW), jnp.float32),
    ],
)
def _sc_deg(dst_hbm, ones_hbm, zeros_hbm, out_hbm, dstv, onesv, shared):
    c = lax.axis_index("c")
    s = lax.axis_index("s")
    w = c * NS + s
    pltpu.sync_copy(dst_hbm.at[w], dstv)
    pltpu.sync_copy(ones_hbm, onesv)
    sl = pl.ds(s * ROWS_PER_TILE, ROWS_PER_TILE)
    pltpu.sync_copy(zeros_hbm, shared.at[sl])
    plsc.subcore_barrier()

    def body(b, carry):
        pltpu.sync_copy(onesv, shared.at[dstv.at[b]], add=True)
        return carry

    lax.fori_loop(0, NB, body, 0)
    plsc.subcore_barrier()
    pltpu.sync_copy(shared.at[sl], out_hbm.at[c].at[sl])


@functools.partial(
    pl.kernel,
    out_type=[
        jax.ShapeDtypeStruct((NW, NWIN, NB3, B2), jnp.int32),
        jax.ShapeDtypeStruct((NW, NWIN, NB3, B2), jnp.int32),
        jax.ShapeDtypeStruct((NW, 16), jnp.int32),
    ],
    mesh=_SC_MESH,
    scratch_types=[
        pltpu.VMEM((VREGS, 16), jnp.int32),
        pltpu.VMEM((VREGS, 16), jnp.int32),
        pltpu.VMEM((NWIN, NB3, B2), jnp.int32),
        pltpu.VMEM((NWIN, NB3, B2), jnp.int32),
        pltpu.VMEM((16,), jnp.int32),
    ],
    compiler_params=pltpu.CompilerParams(
        needs_layout_passes=False, use_tc_tiling_on_sc=False),
)
def _sc_part(src_hbm, dst_hbm, csrc_hbm, cdst_hbm, cnt_hbm,
             srcst, dstst, lsrc, ldst, cntv):
    c = lax.axis_index("c")
    s = lax.axis_index("s")
    w = c * NS + s
    pltpu.sync_copy(src_hbm.at[w], srcst)
    pltpu.sync_copy(dst_hbm.at[w], dstst)
    iota = lax.iota(jnp.int32, 16)

    def vbody(k, offs):
        sv = srcst[k]
        dv = dstst[k]
        new = []
        for win in range(NWIN):
            m = (dv >= win * WR) & (dv < (win + 1) * WR)
            cum = plsc.cumsum(m.astype(jnp.int32))
            pos = offs[win] + cum - 1
            row = lax.shift_right_logical(pos, B2SH)
            col = lax.bitwise_and(pos, B2 - 1)
            plsc.store_scatter(lsrc.at[win], [row, col], sv, mask=m)
            plsc.store_scatter(ldst.at[win], [row, col], dv - win * WR, mask=m)
            new.append(offs[win] + jnp.max(cum))
        return tuple(new)

    z = jnp.int32(0)
    offs = lax.fori_loop(0, VREGS, vbody, (z,) * NWIN)
    for win in range(NWIN):
        for t in range(4):
            pos = jnp.full((16,), offs[win] + t * 16, jnp.int32) + iota
            row = lax.shift_right_logical(pos, B2SH)
            col = lax.bitwise_and(pos, B2 - 1)
            plsc.store_scatter(lsrc.at[win], [row, col], N + iota)
            plsc.store_scatter(ldst.at[win], [row, col], iota)
    cv = jnp.where(iota == 0, offs[0], 0)
    for win in range(1, NWIN):
        cv = jnp.where(iota == win, offs[win], cv)
    cntv[...] = cv
    pltpu.sync_copy(lsrc, csrc_hbm.at[w])
    pltpu.sync_copy(ldst, cdst_hbm.at[w])
    pltpu.sync_copy(cntv, cnt_hbm.at[w])


@functools.partial(
    pl.kernel,
    out_type=jax.ShapeDtypeStruct((NP, D_H), jnp.float32),
    mesh=_SC_MESH,
    scratch_types=[
        pltpu.VMEM((NB3, B2), jnp.int32),
        pltpu.VMEM((NB3, B2), jnp.int32),
        pltpu.VMEM((16,), jnp.int32),
        pltpu.VMEM((B2, D_H), jnp.float32),
        pltpu.VMEM((B2, D_H), jnp.float32),
        pltpu.VMEM_SHARED((WR, D_H), jnp.float32),
        pltpu.SemaphoreType.DMA,
        pltpu.SemaphoreType.DMA,
    ],
    compiler_params=pltpu.CompilerParams(
        needs_layout_passes=False, use_tc_tiling_on_sc=False),
)
def _sc_scatter(csrc_hbm, cdst_hbm, cnt_hbm, f_hbm, zeros_hbm, out_hbm,
                csv, cdv, cntv, rows0, rows1, shared, sem0, sem1):
    c = lax.axis_index("c")
    s = lax.axis_index("s")
    iota = lax.iota(jnp.int32, 16)
    lstripe = pl.ds(s * SPT, SPT)
    for p in range(NWIN // NC):
        wi = p * NC + c
        pltpu.sync_copy(zeros_hbm, shared.at[lstripe])
        plsc.subcore_barrier()
        for half in range(2):
            wsrc = 2 * s + half
            pltpu.sync_copy(cnt_hbm.at[wsrc], cntv)
            pltpu.sync_copy(csrc_hbm.at[wsrc].at[wi], csv)
            pltpu.sync_copy(cdst_hbm.at[wsrc].at[wi], cdv)
            cnt = jnp.max(jnp.where(iota == wi, cntv[...], 0))
            nb = (cnt + B2 - 1) // B2

            @pl.when(nb > 0)
            def _():
                pltpu.async_copy(f_hbm.at[csv.at[0]], rows0, sem0)

            def body(b, carry):
                def step(rj, rk, sj, sk):
                    pltpu.make_async_copy(f_hbm.at[csv.at[b]], rj, sj).wait()

                    @pl.when(b + 1 < nb)
                    def _():
                        pltpu.async_copy(f_hbm.at[csv.at[b + 1]], rk, sk)

                    pltpu.sync_copy(rj, shared.at[cdv.at[b]], add=True)

                @pl.when(lax.rem(b, 2) == 0)
                def _():
                    step(rows0, rows1, sem0, sem1)

                @pl.when(lax.rem(b, 2) == 1)
                def _():
                    step(rows1, rows0, sem1, sem0)

                return carry

            lax.fori_loop(0, nb, body, 0)
        plsc.subcore_barrier()
        pltpu.sync_copy(shared.at[lstripe],
                        out_hbm.at[pl.ds(wi * WR + s * SPT, SPT)])


def _ln(v, g, b):
    m = jnp.mean(v, axis=-1, keepdims=True)
    var = jnp.mean((v - m) ** 2, axis=-1, keepdims=True)
    return (v - m) * lax.rsqrt(var + 1e-5) * g + b


def _ff_body(x_ref, w1_ref, b1_ref, g1_ref, be1_ref, w2_ref, b2_ref,
             g2_ref, be2_ref, o_ref):
    h = jnp.dot(x_ref[...], w1_ref[...], preferred_element_type=jnp.float32)
    h = h + b1_ref[...]
    h = h * jax.nn.sigmoid(h)
    h = _ln(h, g1_ref[...], be1_ref[...])
    h = jnp.dot(h, w2_ref[...], preferred_element_type=jnp.float32)
    h = h + b2_ref[...]
    o_ref[...] = _ln(h, g2_ref[...], be2_ref[...])


def _tc_ff(x, p):
    full = lambda shape: pl.BlockSpec(shape, lambda i: (0,) * len(shape))
    return pl.pallas_call(
        _ff_body,
        grid=(GRID,),
        in_specs=[
            pl.BlockSpec((NBK, D_IN), lambda i: (i, 0)),
            full((D_IN, INNER)),
            full((1, INNER)), full((1, INNER)), full((1, INNER)),
            full((INNER, D_H)),
            full((1, D_H)), full((1, D_H)), full((1, D_H)),
        ],
        out_specs=pl.BlockSpec((NBK, D_H), lambda i: (i, 0)),
        out_shape=jax.ShapeDtypeStruct((NP, D_H), jnp.float32),
    )(x, p['W1'], p['b1'].reshape(1, -1), p['ln1_g'].reshape(1, -1),
      p['ln1_b'].reshape(1, -1), p['W2'], p['b2'].reshape(1, -1),
      p['ln2_g'].reshape(1, -1), p['ln2_b'].reshape(1, -1))


def _row_mask(col):
    rid = (pl.program_id(0) * NBK
           + lax.broadcasted_iota(jnp.int32, (NBK, 1), 0))
    return jnp.where(rid < N, col, 0.0)


def _prep_body(deg_ref, h_ref, norm_ref, f_ref):
    deg = deg_ref[0, :, 0:1] + deg_ref[1, :, 0:1]
    norm = lax.rsqrt(jnp.maximum(deg, 1.0))
    norm_ref[...] = jnp.broadcast_to(norm, (NBK, CW))
    f_ref[...] = h_ref[...] * _row_mask(norm)


def _tc_prep(deg_parts, h):
    return pl.pallas_call(
        _prep_body,
        grid=(GRID,),
        in_specs=[
            pl.BlockSpec((NC, NBK, CW), lambda i: (0, i, 0)),
            pl.BlockSpec((NBK, D_H), lambda i: (i, 0)),
        ],
        out_specs=[
            pl.BlockSpec((NBK, CW), lambda i: (i, 0)),
            pl.BlockSpec((NBK, D_H), lambda i: (i, 0)),
        ],
        out_shape=[
            jax.ShapeDtypeStruct((NP, CW), jnp.float32),
            jax.ShapeDtypeStruct((NP, D_H), jnp.float32),
        ],
    )(deg_parts, h)


def _gcn_hnew(agg_ref, h_ref, norm_ref, wg_ref, bg_ref, wr_ref, br_ref):
    norm = norm_ref[:, 0:1]
    f = agg_ref[...] * norm
    hn = (jnp.dot(f, wg_ref[...], preferred_element_type=jnp.float32)
          + jnp.dot(h_ref[...], wr_ref[...], preferred_element_type=jnp.float32)
          + bg_ref[...] + br_ref[...])
    return hn, norm


def _gcn_body(agg_ref, h_ref, norm_ref, wg_ref, bg_ref, wr_ref, br_ref,
              h_out, f_out):
    hn, norm = _gcn_hnew(agg_ref, h_ref, norm_ref, wg_ref, bg_ref,
                         wr_ref, br_ref)
    h_out[...] = hn
    f_out[...] = hn * _row_mask(norm)


def _gcn_final_body(agg_ref, h_ref, norm_ref, wg_ref, bg_ref, wr_ref, br_ref,
                    o_ref):
    hn, _ = _gcn_hnew(agg_ref, h_ref, norm_ref, wg_ref, bg_ref,
                      wr_ref, br_ref)
    rid = (pl.program_id(0) * NBK
           + lax.broadcasted_iota(jnp.int32, (NBK, 1), 0))
    hn = jnp.where(rid < N, hn, -jnp.inf)

    @pl.when(pl.program_id(0) == 0)
    def _():
        o_ref[...] = jnp.full((1, D_H), -jnp.inf, dtype=jnp.float32)

    o_ref[...] = jnp.maximum(o_ref[...], jnp.max(hn, axis=0, keepdims=True))


def _gcn_specs(final):
    in_specs = [
        pl.BlockSpec((NBK, D_H), lambda i: (i, 0)),
        pl.BlockSpec((NBK, D_H), lambda i: (i, 0)),
        pl.BlockSpec((NBK, CW), lambda i: (i, 0)),
        pl.BlockSpec((D_H, D_H), lambda i: (0, 0)),
        pl.BlockSpec((1, D_H), lambda i: (0, 0)),
        pl.BlockSpec((D_H, D_H), lambda i: (0, 0)),
        pl.BlockSpec((1, D_H), lambda i: (0, 0)),
    ]
    if final:
        return in_specs, pl.BlockSpec((1, D_H), lambda i: (0, 0)), \
            jax.ShapeDtypeStruct((1, D_H), jnp.float32)
    return in_specs, [
        pl.BlockSpec((NBK, D_H), lambda i: (i, 0)),
        pl.BlockSpec((NBK, D_H), lambda i: (i, 0)),
    ], [
        jax.ShapeDtypeStruct((NP, D_H), jnp.float32),
        jax.ShapeDtypeStruct((NP, D_H), jnp.float32),
    ]


def _tc_gcn(agg, h, norm128, wg, bg, wr, br, final):
    in_specs, out_specs, out_shape = _gcn_specs(final)
    body = _gcn_final_body if final else _gcn_body
    return pl.pallas_call(
        body,
        grid=(GRID,),
        in_specs=in_specs,
        out_specs=out_specs,
        out_shape=out_shape,
    )(agg, h, norm128, wg, bg.reshape(1, -1), wr, br.reshape(1, -1))


def kernel(x, edge_index, params):
    ei = edge_index.astype(jnp.int32)
    src = ei[0]
    dst = ei[1]
    dst3 = dst.reshape(NW, NB, B)
    padn = EP - E
    pad_idx = jnp.arange(padn, dtype=jnp.int32)
    src_p = jnp.concatenate([src, N + pad_idx % (NP - N)]).reshape(
        NW, VREGS, 16)
    dst_p = jnp.concatenate([dst, pad_idx % CW]).reshape(NW, VREGS, 16)
    x = jnp.pad(x, ((0, NP - N), (0, 0)))
    ones128 = jnp.ones((B, CW), jnp.float32)
    zeros_deg = jnp.zeros((ROWS_PER_TILE, CW), jnp.float32)
    zeros_win = jnp.zeros((SPT, D_H), jnp.float32)

    csrc, cdst, cnt = _sc_part(src_p, dst_p)
    zeros_deg = zeros_deg + (cnt[0, 0] * 0).astype(jnp.float32)
    deg_parts = _sc_deg(dst3, ones128, zeros_deg)
    h = _tc_ff(x, params)
    norm128, f = _tc_prep(deg_parts, h)
    out = None
    for i in range(4):
        agg = _sc_scatter(csrc, cdst, cnt, f, zeros_win)
        res = _tc_gcn(agg, h, norm128,
                      params['gcn%d_W' % i], params['gcn%d_b' % i],
                      params['res%d_W' % i], params['res%d_b' % i],
                      final=(i == 3))
        if i < 3:
            h, f = res
        else:
            out = res
    return out


# speedup vs baseline: 1.0968x; 1.0936x over previous
import functools

import jax
import jax.numpy as jnp
from jax import lax
from jax.experimental import pallas as pl
from jax.experimental.pallas import tpu as pltpu
from jax.experimental.pallas import tpu_sc as plsc

N = 10000
NP = 10240
E = 160000
D_IN = 256
INNER = 1024
D_H = 512
FC = 4
CW = D_H // FC
NC, NS = 2, 16
NW = NC * NS
EPW = E // NW
B = 125
NB = EPW // B
ROWS_PER_TILE = NP // NS
ZB = 128
NZ = ROWS_PER_TILE // ZB

NBK = 1024
GRID = NP // NBK

_SC_MESH = plsc.VectorSubcoreMesh(core_axis_name="c", subcore_axis_name="s")


@functools.partial(
    pl.kernel,
    out_type=jax.ShapeDtypeStruct((NC, NP, CW), jnp.float32),
    mesh=_SC_MESH,
    scratch_types=[
        pltpu.VMEM((NB, B), jnp.int32),
        pltpu.VMEM((B, CW), jnp.float32),
        pltpu.VMEM_SHARED((NP, CW), jnp.float32),
    ],
)
def _sc_deg(dst_hbm, ones_hbm, zeros_hbm, out_hbm, dstv, onesv, shared):
    c = lax.axis_index("c")
    s = lax.axis_index("s")
    w = c * NS + s
    pltpu.sync_copy(dst_hbm.at[w], dstv)
    pltpu.sync_copy(ones_hbm, onesv)
    base = s * ROWS_PER_TILE
    sl = pl.ds(base, ROWS_PER_TILE)
    pltpu.sync_copy(zeros_hbm, shared.at[sl])
    plsc.subcore_barrier()

    def body(b, carry):
        pltpu.sync_copy(onesv, shared.at[dstv.at[b]], add=True)
        return carry

    lax.fori_loop(0, NB, body, 0)
    plsc.subcore_barrier()
    pltpu.sync_copy(shared.at[sl], out_hbm.at[c].at[sl])


@functools.partial(
    pl.kernel,
    out_type=jax.ShapeDtypeStruct((NC, FC, NP, CW), jnp.float32),
    mesh=_SC_MESH,
    scratch_types=[
        pltpu.VMEM((NB, B), jnp.int32),
        pltpu.VMEM((NB, B), jnp.int32),
        pltpu.VMEM((B, CW), jnp.float32),
        pltpu.VMEM((B, CW), jnp.float32),
        pltpu.VMEM_SHARED((NP, CW), jnp.float32),
        pltpu.SemaphoreType.DMA,
        pltpu.SemaphoreType.DMA,
    ],
)
def _sc_scatter(src_hbm, dst_hbm, f_hbm, zeros_hbm, out_hbm,
                srcv, dstv, rows0, rows1, shared, sem0, sem1):
    c = lax.axis_index("c")
    s = lax.axis_index("s")
    w = c * NS + s
    pltpu.sync_copy(src_hbm.at[w], srcv)
    pltpu.sync_copy(dst_hbm.at[w], dstv)
    base = s * ROWS_PER_TILE
    stripe = pl.ds(base, ROWS_PER_TILE)
    rows = (rows0, rows1)
    sems = (sem0, sem1)
    for fc in range(FC):
        pltpu.sync_copy(zeros_hbm, shared.at[stripe])
        plsc.subcore_barrier()
        pltpu.async_copy(f_hbm.at[fc].at[srcv.at[0]], rows0, sem0)

        def body(i, carry):
            for j in range(2):
                b = 2 * i + j
                pltpu.make_async_copy(
                    f_hbm.at[fc].at[srcv.at[b]], rows[j], sems[j]).wait()

                @pl.when(b + 1 < NB)
                def _():
                    pltpu.async_copy(
                        f_hbm.at[fc].at[srcv.at[b + 1]], rows[j ^ 1],
                        sems[j ^ 1])

                pltpu.sync_copy(rows[j], shared.at[dstv.at[b]], add=True)
            return carry

        lax.fori_loop(0, NB // 2, body, 0)
        plsc.subcore_barrier()
        pltpu.sync_copy(shared.at[stripe], out_hbm.at[c].at[fc].at[stripe])


def _ln(v, g, b):
    m = jnp.mean(v, axis=-1, keepdims=True)
    var = jnp.mean((v - m) ** 2, axis=-1, keepdims=True)
    return (v - m) * lax.rsqrt(var + 1e-5) * g + b


def _ff_body(x_ref, w1_ref, b1_ref, g1_ref, be1_ref, w2_ref, b2_ref,
             g2_ref, be2_ref, o_ref):
    h = jnp.dot(x_ref[...], w1_ref[...], preferred_element_type=jnp.float32)
    h = h + b1_ref[...]
    h = h * jax.nn.sigmoid(h)
    h = _ln(h, g1_ref[...], be1_ref[...])
    h = jnp.dot(h, w2_ref[...], preferred_element_type=jnp.float32)
    h = h + b2_ref[...]
    o_ref[...] = _ln(h, g2_ref[...], be2_ref[...])


def _tc_ff(x, p):
    full = lambda shape: pl.BlockSpec(shape, lambda i: (0,) * len(shape))
    return pl.pallas_call(
        _ff_body,
        grid=(GRID,),
        in_specs=[
            pl.BlockSpec((NBK, D_IN), lambda i: (i, 0)),
            full((D_IN, INNER)),
            full((1, INNER)), full((1, INNER)), full((1, INNER)),
            full((INNER, D_H)),
            full((1, D_H)), full((1, D_H)), full((1, D_H)),
        ],
        out_specs=pl.BlockSpec((NBK, D_H), lambda i: (i, 0)),
        out_shape=jax.ShapeDtypeStruct((NP, D_H), jnp.float32),
    )(x, p['W1'], p['b1'].reshape(1, -1), p['ln1_g'].reshape(1, -1),
      p['ln1_b'].reshape(1, -1), p['W2'], p['b2'].reshape(1, -1),
      p['ln2_g'].reshape(1, -1), p['ln2_b'].reshape(1, -1))


def _prep_body(deg_ref, h_ref, norm_ref, f_ref):
    deg = deg_ref[0, :, 0:1] + deg_ref[1, :, 0:1]
    norm = lax.rsqrt(jnp.maximum(deg, 1.0))
    norm_ref[...] = jnp.broadcast_to(norm, (NBK, CW))
    for fc in range(FC):
        f_ref[fc] = h_ref[:, fc * CW:(fc + 1) * CW] * norm


def _tc_prep(deg_parts, h):
    return pl.pallas_call(
        _prep_body,
        grid=(GRID,),
        in_specs=[
            pl.BlockSpec((NC, NBK, CW), lambda i: (0, i, 0)),
            pl.BlockSpec((NBK, D_H), lambda i: (i, 0)),
        ],
        out_specs=[
            pl.BlockSpec((NBK, CW), lambda i: (i, 0)),
            pl.BlockSpec((FC, NBK, CW), lambda i: (0, i, 0)),
        ],
        out_shape=[
            jax.ShapeDtypeStruct((NP, CW), jnp.float32),
            jax.ShapeDtypeStruct((FC, NP, CW), jnp.float32),
        ],
    )(deg_parts, h)


def _gcn_hnew(part_ref, h_ref, norm_ref, wg_ref, bg_ref, wr_ref, br_ref):
    norm = norm_ref[:, 0:1]
    agg = jnp.concatenate(
        [part_ref[0, fc] + part_ref[1, fc] for fc in range(FC)], axis=-1)
    f = agg * norm
    hn = (jnp.dot(f, wg_ref[...], preferred_element_type=jnp.float32)
          + jnp.dot(h_ref[...], wr_ref[...],
                    preferred_element_type=jnp.float32)
          + bg_ref[...] + br_ref[...])
    return hn, norm


def _gcn_body(part_ref, h_ref, norm_ref, wg_ref, bg_ref, wr_ref, br_ref,
              h_out, f_out):
    hn, norm = _gcn_hnew(part_ref, h_ref, norm_ref, wg_ref, bg_ref,
                         wr_ref, br_ref)
    h_out[...] = hn
    for fc in range(FC):
        f_out[fc] = hn[:, fc * CW:(fc + 1) * CW] * norm


def _tc_gcn(part, h, norm128, wg, bg, wr, br):
    return pl.pallas_call(
        _gcn_body,
        grid=(GRID,),
        in_specs=[
            pl.BlockSpec((NC, FC, NBK, CW), lambda i: (0, 0, i, 0)),
            pl.BlockSpec((NBK, D_H), lambda i: (i, 0)),
            pl.BlockSpec((NBK, CW), lambda i: (i, 0)),
            pl.BlockSpec((D_H, D_H), lambda i: (0, 0)),
            pl.BlockSpec((1, D_H), lambda i: (0, 0)),
            pl.BlockSpec((D_H, D_H), lambda i: (0, 0)),
            pl.BlockSpec((1, D_H), lambda i: (0, 0)),
        ],
        out_specs=[
            pl.BlockSpec((NBK, D_H), lambda i: (i, 0)),
            pl.BlockSpec((FC, NBK, CW), lambda i: (0, i, 0)),
        ],
        out_shape=[
            jax.ShapeDtypeStruct((NP, D_H), jnp.float32),
            jax.ShapeDtypeStruct((FC, NP, CW), jnp.float32),
        ],
    )(part, h, norm128, wg, bg.reshape(1, -1), wr, br.reshape(1, -1))


def _gcn_final_body(part_ref, h_ref, norm_ref, wg_ref, bg_ref, wr_ref,
                    br_ref, o_ref):
    hn, _ = _gcn_hnew(part_ref, h_ref, norm_ref, wg_ref, bg_ref,
                      wr_ref, br_ref)
    rid = (pl.program_id(0) * NBK
           + lax.broadcasted_iota(jnp.int32, (NBK, 1), 0))
    hn = jnp.where(rid < N, hn, -jnp.inf)

    @pl.when(pl.program_id(0) == 0)
    def _():
        o_ref[...] = jnp.full((1, D_H), -jnp.inf, dtype=jnp.float32)

    o_ref[...] = jnp.maximum(o_ref[...], jnp.max(hn, axis=0, keepdims=True))


def _tc_gcn_final(part, h, norm128, wg, bg, wr, br):
    return pl.pallas_call(
        _gcn_final_body,
        grid=(GRID,),
        in_specs=[
            pl.BlockSpec((NC, FC, NBK, CW), lambda i: (0, 0, i, 0)),
            pl.BlockSpec((NBK, D_H), lambda i: (i, 0)),
            pl.BlockSpec((NBK, CW), lambda i: (i, 0)),
            pl.BlockSpec((D_H, D_H), lambda i: (0, 0)),
            pl.BlockSpec((1, D_H), lambda i: (0, 0)),
            pl.BlockSpec((D_H, D_H), lambda i: (0, 0)),
            pl.BlockSpec((1, D_H), lambda i: (0, 0)),
        ],
        out_specs=pl.BlockSpec((1, D_H), lambda i: (0, 0)),
        out_shape=jax.ShapeDtypeStruct((1, D_H), jnp.float32),
    )(part, h, norm128, wg, bg.reshape(1, -1), wr, br.reshape(1, -1))


def kernel(x, edge_index, params):
    ei = edge_index.astype(jnp.int32)
    src3 = ei[0].reshape(NW, NB, B)
    dst3 = ei[1].reshape(NW, NB, B)
    x = jnp.pad(x, ((0, NP - N), (0, 0)))
    ones128 = jnp.ones((B, CW), jnp.float32)
    zeros128 = jnp.zeros((ROWS_PER_TILE, CW), jnp.float32)

    deg_parts = _sc_deg(dst3, ones128, zeros128)
    h = _tc_ff(x, params)
    norm128, f = _tc_prep(deg_parts, h)
    out = None
    for i in range(4):
        part = _sc_scatter(src3, dst3, f, zeros128)
        args = (part, h, norm128,
                params['gcn%d_W' % i], params['gcn%d_b' % i],
                params['res%d_W' % i], params['res%d_b' % i])
        if i < 3:
            h, f = _tc_gcn(*args)
        else:
            out = _tc_gcn_final(*args)
    return out
